# Initial kernel scaffold; baseline (speedup 1.0000x reference)
import jax
import jax.numpy as jnp
from jax.experimental import pallas as pl


def kernel(x, gamma, beta, Wg, bg):
    raise NotImplementedError("write your pallas kernel here")


# speedup vs baseline: 4.7702x; 4.7702x over previous
import jax
import jax.numpy as jnp
from jax import lax
from jax.experimental import pallas as pl
from jax.experimental.pallas import tpu as pltpu

_B, _T, _N, _E = 16, 96, 2048, 16
_EPS = 1e-5


def _body(x_ref, gamma_ref, beta_ref, Wg_ref, bg_ref,
          out_ref, msp_ref, ssp_ref, gb_ref, bb_ref, md_ref, sd_ref):
    xb = x_ref[0]
    m1 = jnp.mean(xb, axis=0, keepdims=True)
    m2 = jnp.mean(xb * xb, axis=0, keepdims=True)
    mu_d = jnp.mean(m1)
    var_d = jnp.mean(m2) - mu_d * mu_d
    std_d = jnp.sqrt(var_d + _EPS)
    mean_sp = (m1 - mu_d) / std_d
    var_sp = (m2 - m1 * m1) / (var_d + _EPS)
    std_sp = jnp.sqrt(var_sp + _EPS)
    s = 1.0 / (std_d * std_sp)

    wg = Wg_ref[...]
    G = jnp.dot(wg, xb, preferred_element_type=jnp.float32)
    W1 = jnp.sum(wg, axis=1, keepdims=True)
    scores = s * (G - W1 * m1) + bg_ref[...]

    best = scores[0:1, :]
    idx = jnp.zeros((1, _N), jnp.int32)
    for e in range(1, _E):
        row = scores[e:e + 1, :]
        upd = row > best
        best = jnp.where(upd, row, best)
        idx = jnp.where(upd, e, idx)

    gam = gamma_ref[...]
    bet = beta_ref[...]
    gsel = jnp.full((1, _N), gam[0, 0])
    bsel = jnp.full((1, _N), bet[0, 0])
    for e in range(1, _E):
        pe = idx == e
        gsel = jnp.where(pe, gam[e, 0], gsel)
        bsel = jnp.where(pe, bet[e, 0], bsel)

    out_ref[0] = (xb - m1) * (s * gsel) + bsel
    msp_ref[0] = mean_sp
    ssp_ref[0] = std_sp
    gb_ref[0] = gsel
    bb_ref[0] = bsel
    md_ref[...] = jnp.full((1, 1, 1), mu_d)
    sd_ref[...] = jnp.full((1, 1, 1), std_d)


def kernel(x, gamma, beta, Wg, bg):
    x3 = x.reshape(_B, _T, _N)
    bg2 = bg.reshape(_E, 1)
    rep = pl.BlockSpec((_E, 1), lambda b: (0, 0))
    wspec = pl.BlockSpec((_E, _T), lambda b: (0, 0))
    out, msp, ssp, gb, bb, md, sd = pl.pallas_call(
        _body,
        grid=(_B,),
        in_specs=[
            pl.BlockSpec((1, _T, _N), lambda b: (b, 0, 0)),
            rep, rep, wspec, rep,
        ],
        out_specs=[
            pl.BlockSpec((1, _T, _N), lambda b: (b, 0, 0)),
            pl.BlockSpec((1, 1, _N), lambda b: (b, 0, 0)),
            pl.BlockSpec((1, 1, _N), lambda b: (b, 0, 0)),
            pl.BlockSpec((1, 1, _N), lambda b: (b, 0, 0)),
            pl.BlockSpec((1, 1, _N), lambda b: (b, 0, 0)),
            pl.BlockSpec((1, 1, 1), lambda b: (b, 0, 0)),
            pl.BlockSpec((1, 1, 1), lambda b: (b, 0, 0)),
        ],
        out_shape=[
            jax.ShapeDtypeStruct((_B, _T, _N), jnp.float32),
            jax.ShapeDtypeStruct((_B, 1, _N), jnp.float32),
            jax.ShapeDtypeStruct((_B, 1, _N), jnp.float32),
            jax.ShapeDtypeStruct((_B, 1, _N), jnp.float32),
            jax.ShapeDtypeStruct((_B, 1, _N), jnp.float32),
            jax.ShapeDtypeStruct((_B, 1, 1), jnp.float32),
            jax.ShapeDtypeStruct((_B, 1, 1), jnp.float32),
        ],
    )(x3, gamma, beta, Wg, bg2)
    return (out.reshape(_B, _T, _N, 1),
            gb.reshape(_B, 1, _N, 1),
            bb.reshape(_B, 1, _N, 1),
            msp.reshape(_B, 1, _N, 1),
            ssp.reshape(_B, 1, _N, 1),
            md.reshape(_B, 1, 1, 1),
            sd.reshape(_B, 1, 1, 1))


# speedup vs baseline: 10.5279x; 2.2070x over previous
import jax
import jax.numpy as jnp
from jax import lax
from jax.experimental import pallas as pl
from jax.experimental.pallas import tpu as pltpu

_B, _T, _N, _E = 16, 96, 2048, 16
_G = _N // 128
_R = _T * _G
_EPS = 1e-5


def _body(x_ref, gamma_ref, beta_ref, Wg_ref, bg_ref,
          out_ref, msp_ref, ssp_ref, gb_ref, bb_ref, md_ref, sd_ref):
    x4 = x_ref[0].reshape(_T, _G, 128)
    m1 = jnp.mean(x4, axis=0)
    m2 = jnp.mean(x4 * x4, axis=0)
    mu_d = jnp.mean(m1)
    var_d = jnp.mean(m2) - mu_d * mu_d
    std_d = jnp.sqrt(var_d + _EPS)
    mean_sp = (m1 - mu_d) / std_d
    var_sp = (m2 - m1 * m1) / (var_d + _EPS)
    std_sp = jnp.sqrt(var_sp + _EPS)
    s = 1.0 / (std_d * std_sp)

    wg = Wg_ref[...]
    W1 = jnp.sum(wg, axis=1, keepdims=True)
    bgv = bg_ref[...]

    idx_g, best_g = [], []
    for g in range(_G):
        Gg = jnp.dot(wg, x4[:, g, :],
                     preferred_element_type=jnp.float32)
        sc = s[g:g + 1, :] * (Gg - W1 * m1[g:g + 1, :]) + bgv
        best = sc[0:1, :]
        idx = jnp.zeros((1, 128), jnp.int32)
        for e in range(1, _E):
            row = sc[e:e + 1, :]
            upd = row > best
            best = jnp.where(upd, row, best)
            idx = jnp.where(upd, e, idx)
        idx_g.append(idx)
    idx = jnp.concatenate(idx_g, axis=0)

    gam = gamma_ref[...]
    bet = beta_ref[...]
    gsel = jnp.full((_G, 128), gam[0, 0])
    bsel = jnp.full((_G, 128), bet[0, 0])
    for e in range(1, _E):
        pe = idx == e
        gsel = jnp.where(pe, gam[e, 0], gsel)
        bsel = jnp.where(pe, bet[e, 0], bsel)

    scale = s * gsel
    out4 = (x4 - m1[None]) * scale[None] + bsel[None]
    out_ref[0] = out4.reshape(_R, 128)
    msp_ref[0] = mean_sp
    ssp_ref[0] = std_sp
    gb_ref[0] = gsel
    bb_ref[0] = bsel
    md_ref[...] = jnp.full((1, 1, 1), mu_d)
    sd_ref[...] = jnp.full((1, 1, 1), std_d)


def kernel(x, gamma, beta, Wg, bg):
    x2 = x.reshape(_B, _R, 128)
    bg2 = bg.reshape(_E, 1)
    rep = pl.BlockSpec((_E, 1), lambda b: (0, 0))
    wspec = pl.BlockSpec((_E, _T), lambda b: (0, 0))
    out, msp, ssp, gb, bb, md, sd = pl.pallas_call(
        _body,
        grid=(_B,),
        in_specs=[
            pl.BlockSpec((1, _R, 128), lambda b: (b, 0, 0)),
            rep, rep, wspec, rep,
        ],
        out_specs=[
            pl.BlockSpec((1, _R, 128), lambda b: (b, 0, 0)),
            pl.BlockSpec((1, _G, 128), lambda b: (b, 0, 0)),
            pl.BlockSpec((1, _G, 128), lambda b: (b, 0, 0)),
            pl.BlockSpec((1, _G, 128), lambda b: (b, 0, 0)),
            pl.BlockSpec((1, _G, 128), lambda b: (b, 0, 0)),
            pl.BlockSpec((1, 1, 1), lambda b: (b, 0, 0)),
            pl.BlockSpec((1, 1, 1), lambda b: (b, 0, 0)),
        ],
        out_shape=[
            jax.ShapeDtypeStruct((_B, _R, 128), jnp.float32),
            jax.ShapeDtypeStruct((_B, _G, 128), jnp.float32),
            jax.ShapeDtypeStruct((_B, _G, 128), jnp.float32),
            jax.ShapeDtypeStruct((_B, _G, 128), jnp.float32),
            jax.ShapeDtypeStruct((_B, _G, 128), jnp.float32),
            jax.ShapeDtypeStruct((_B, 1, 1), jnp.float32),
            jax.ShapeDtypeStruct((_B, 1, 1), jnp.float32),
        ],
    )(x2, gamma, beta, Wg, bg2)
    return (out.reshape(_B, _T, _N, 1),
            gb.reshape(_B, 1, _N, 1),
            bb.reshape(_B, 1, _N, 1),
            msp.reshape(_B, 1, _N, 1),
            ssp.reshape(_B, 1, _N, 1),
            md.reshape(_B, 1, 1, 1),
            sd.reshape(_B, 1, 1, 1))
